# baseline (speedup 1.0000x reference)
import functools

import jax
import jax.numpy as jnp
from jax import lax
from jax.experimental import pallas as pl
from jax.experimental.pallas import tpu as pltpu
from jax.experimental.pallas import tpu_sc as plsc

S = 64.0
M = 0.4
_MS = M * S

_B = 1024
_V = 100000

_SC_ROWS = 320
_TC_ROWS = _B - _SC_ROWS

_BLOCK_COLS = 2048


def _cosface_block(labels_ref, logits_ref, out_ref):
    pid = pl.program_id(0)
    block = logits_ref[...]
    rows, cols = block.shape
    col_ids = jax.lax.broadcasted_iota(jnp.int32, (rows, cols), 1) + pid * cols
    mask = col_ids == labels_ref[...]
    out_ref[...] = block * S - jnp.where(mask, M * S, 0.0)


def _tc_call(logits, labels2d):
    grid = (pl.cdiv(_V, _BLOCK_COLS),)
    return pl.pallas_call(
        _cosface_block,
        grid=grid,
        in_specs=[
            pl.BlockSpec((_TC_ROWS, 1), lambda i: (0, 0)),
            pl.BlockSpec((_TC_ROWS, _BLOCK_COLS), lambda i: (0, i)),
        ],
        out_specs=pl.BlockSpec((_TC_ROWS, _BLOCK_COLS), lambda i: (0, i)),
        out_shape=jax.ShapeDtypeStruct((_B, _V), logits.dtype),
    )(labels2d, logits)


_NW = 32
_ROWS_PER_W = _SC_ROWS // _NW
_CHUNK = 20000
_CHUNKS_PER_ROW = _V // _CHUNK
_VECS = _CHUNK // 16
_T = _ROWS_PER_W * _CHUNKS_PER_ROW
_NB = 2
_G = _T // _NB


def _sc_body(logits_hbm, labels_hbm, out_hbm, in_bufs, out_bufs, labels_v,
             in_sems, out_sems):
    cid = lax.axis_index("c")
    sid = lax.axis_index("s")
    wid = sid * 2 + cid
    r0 = _TC_ROWS + wid * _ROWS_PER_W
    base = r0 * _V
    obase = wid * _ROWS_PER_W * _V

    lstart = (r0 // 8) * 8
    lshift = r0 - lstart
    pltpu.sync_copy(labels_hbm.at[pl.ds(lstart, _ROWS_PER_W + 14)],
                    labels_v.at[pl.ds(0, _ROWS_PER_W + 14)])

    def in_copy(t, b):
        return pltpu.make_async_copy(
            logits_hbm.at[pl.ds(base + t * _CHUNK, _CHUNK)],
            in_bufs[b], in_sems[b])

    def out_copy(t, b):
        return pltpu.make_async_copy(
            out_bufs[b], out_hbm.at[pl.ds(obase + t * _CHUNK, _CHUNK)],
            out_sems[b])

    for b in range(_NB):
        in_copy(b, b).start()

    def outer(g, carry):
        for b in range(_NB):
            t = g * _NB + b
            in_copy(t, b).wait()

            @pl.when(g > 0)
            def _drain():
                out_copy(t - _NB, b).wait()

            src = in_bufs[b]
            dst = out_bufs[b]

            def vec_step(i, c):
                dst[pl.ds(i * 16, 16)] = src[pl.ds(i * 16, 16)] * S
                return c

            lax.fori_loop(0, _VECS, vec_step, 0, unroll=8)

            r = t // _CHUNKS_PER_ROW
            c0 = (t - r * _CHUNKS_PER_ROW) * _CHUNK
            lab = labels_v[pl.ds(lshift + r, 16)][0]
            col = lab - c0

            @pl.when(jnp.logical_and(col >= 0, col < _CHUNK))
            def _fix():
                vbase = (col // 16) * 16
                lane = col - vbase
                iota = lax.iota(jnp.int32, 16)
                vec = dst[pl.ds(vbase, 16)]
                dst[pl.ds(vbase, 16)] = vec - jnp.where(iota == lane, _MS, 0.0)

            out_copy(t, b).start()

            @pl.when(g < _G - 1)
            def _prefetch():
                in_copy(t + _NB, b).start()

        return carry

    lax.fori_loop(0, _G, outer, 0)

    for b in range(_NB):
        out_copy(_T - _NB + b, b).wait()


def _sc_call(flat_logits, labels32):
    mesh = plsc.VectorSubcoreMesh(core_axis_name="c", subcore_axis_name="s")
    run = pl.kernel(
        _sc_body,
        out_type=jax.ShapeDtypeStruct((_SC_ROWS * _V,), jnp.float32),
        mesh=mesh,
        scratch_types=[
            [pltpu.VMEM((_CHUNK,), jnp.float32) for _ in range(_NB)],
            [pltpu.VMEM((_CHUNK,), jnp.float32) for _ in range(_NB)],
            pltpu.VMEM((_ROWS_PER_W + 14 + 16,), jnp.int32),
            [pltpu.SemaphoreType.DMA for _ in range(_NB)],
            [pltpu.SemaphoreType.DMA for _ in range(_NB)],
        ],
    )
    return run(flat_logits, labels32)


@jax.jit
def kernel(logits, labels):
    B, V = logits.shape
    labels32 = labels.astype(jnp.int32)
    labels2d = labels32.reshape(B, 1)
    flat = logits.reshape(B * V)
    sc_out = _sc_call(flat, labels32).reshape(_SC_ROWS, _V)
    tc_out = _tc_call(logits, labels2d)
    return lax.dynamic_update_slice(tc_out, sc_out, (_TC_ROWS, 0))


# speedup vs baseline: 1.8381x; 1.8381x over previous
import functools

import jax
import jax.numpy as jnp
from jax.experimental import pallas as pl

S = 64.0
M = 0.4

_BLOCK_ROWS = 256
_BLOCK_COLS = 8192


def _cosface_block(labels_ref, logits_ref, out_ref):
    j = pl.program_id(1)
    block = logits_ref[...]
    rows, cols = block.shape
    col_ids = jax.lax.broadcasted_iota(jnp.int32, (rows, cols), 1) + j * cols
    mask = col_ids == labels_ref[...]
    out_ref[...] = block * S - jnp.where(mask, M * S, 0.0)


@jax.jit
def kernel(logits, labels):
    B, V = logits.shape
    labels2d = labels.astype(jnp.int32).reshape(B, 1)
    grid = (B // _BLOCK_ROWS, pl.cdiv(V, _BLOCK_COLS))
    return pl.pallas_call(
        _cosface_block,
        grid=grid,
        in_specs=[
            pl.BlockSpec((_BLOCK_ROWS, 1), lambda i, j: (i, 0)),
            pl.BlockSpec((_BLOCK_ROWS, _BLOCK_COLS), lambda i, j: (i, j)),
        ],
        out_specs=pl.BlockSpec((_BLOCK_ROWS, _BLOCK_COLS), lambda i, j: (i, j)),
        out_shape=jax.ShapeDtypeStruct((B, V), logits.dtype),
    )(labels2d, logits)


# speedup vs baseline: 1.8416x; 1.0019x over previous
import functools

import jax
import jax.numpy as jnp
from jax.experimental import pallas as pl

S = 64.0
M = 0.4

_BLOCK_ROWS = 512
_BLOCK_COLS = 4096


def _cosface_block(labels_ref, logits_ref, out_ref):
    j = pl.program_id(1)
    block = logits_ref[...]
    rows, cols = block.shape
    col_ids = jax.lax.broadcasted_iota(jnp.int32, (rows, cols), 1) + j * cols
    mask = col_ids == labels_ref[...]
    out_ref[...] = block * S - jnp.where(mask, M * S, 0.0)


@jax.jit
def kernel(logits, labels):
    B, V = logits.shape
    labels2d = labels.astype(jnp.int32).reshape(B, 1)
    grid = (B // _BLOCK_ROWS, pl.cdiv(V, _BLOCK_COLS))
    return pl.pallas_call(
        _cosface_block,
        grid=grid,
        in_specs=[
            pl.BlockSpec((_BLOCK_ROWS, 1), lambda i, j: (i, 0)),
            pl.BlockSpec((_BLOCK_ROWS, _BLOCK_COLS), lambda i, j: (i, j)),
        ],
        out_specs=pl.BlockSpec((_BLOCK_ROWS, _BLOCK_COLS), lambda i, j: (i, j)),
        out_shape=jax.ShapeDtypeStruct((B, V), logits.dtype),
    )(labels2d, logits)
